# Initial kernel scaffold; baseline (speedup 1.0000x reference)
import jax
import jax.numpy as jnp
from jax.experimental import pallas as pl


def kernel(input_scores, alignments, alignment_weights, unflatten, unflatten_mask):
    raise NotImplementedError("write your pallas kernel here")


# speedup vs baseline: 148.6195x; 148.6195x over previous
import functools

import jax
import jax.numpy as jnp
from jax import lax
from jax.experimental import pallas as pl
from jax.experimental.pallas import tpu as pltpu
from jax.experimental.pallas import tpu_sc as plsc

B = 16
L = 4096
NC = 32768
A = 64
C = 2048

NUM_CORES = 2
NUM_SUBCORES = 16
NW = NUM_CORES * NUM_SUBCORES
ROWS_PER_TILE = NC // NW
CHUNK_ROWS = 128
CHUNK_ELEMS = CHUNK_ROWS * A
NUM_CHUNKS = ROWS_PER_TILE // CHUNK_ROWS
TABLE = B * L

_mesh = plsc.VectorSubcoreMesh(
    core_axis_name="c", subcore_axis_name="s",
    num_cores=NUM_CORES, num_subcores=NUM_SUBCORES)
_params = pltpu.CompilerParams(needs_layout_passes=False)


def _wid():
    return lax.axis_index("s") * NUM_CORES + lax.axis_index("c")


@functools.partial(
    pl.kernel,
    out_type=jax.ShapeDtypeStruct((NC,), jnp.float32),
    mesh=_mesh,
    scratch_types=[
        pltpu.VMEM((TABLE,), jnp.float32),
        pltpu.VMEM((CHUNK_ELEMS,), jnp.int32),
        pltpu.VMEM((CHUNK_ELEMS,), jnp.int32),
        pltpu.VMEM((CHUNK_ELEMS,), jnp.float32),
        pltpu.VMEM((CHUNK_ELEMS,), jnp.float32),
        pltpu.VMEM((ROWS_PER_TILE,), jnp.float32),
        pltpu.SemaphoreType.DMA,
        pltpu.SemaphoreType.DMA,
        pltpu.SemaphoreType.DMA,
    ],
    compiler_params=_params,
)
def _score_rows(tbl_hbm, ali_hbm, w_hbm, out_hbm,
                table_v, ib0, ib1, wb0, wb1, out_v, sem_t, sem0, sem1):
    wid = _wid()
    row0 = wid * ROWS_PER_TILE
    e0 = row0 * A

    tcp = pltpu.make_async_copy(tbl_hbm, table_v, sem_t)
    tcp.start()

    ibufs = (ib0, ib1)
    wbufs = (wb0, wb1)
    sems = (sem0, sem1)

    def chunk_copies(ch):
        b = ch % 2
        src = pl.ds(e0 + ch * CHUNK_ELEMS, CHUNK_ELEMS)
        return (pltpu.make_async_copy(ali_hbm.at[src], ibufs[b], sems[b]),
                pltpu.make_async_copy(w_hbm.at[src], wbufs[b], sems[b]))

    for cp in chunk_copies(0):
        cp.start()
    tcp.wait()

    ioff = lax.iota(jnp.int32, 16) * A

    for ch in range(NUM_CHUNKS):
        for cp in chunk_copies(ch):
            cp.wait()
        if ch + 1 < NUM_CHUNKS:
            for cp in chunk_copies(ch + 1):
                cp.start()
        ib = ibufs[ch % 2]
        wb = wbufs[ch % 2]

        def group(g, _, ib=ib, wb=wb, ch=ch):
            base = g * (16 * A)

            def step(a, acc):
                addr = ioff + (base + a)
                iv = plsc.load_gather(ib, [addr])
                wv = plsc.load_gather(wb, [addr])
                tv = plsc.load_gather(table_v, [iv])
                return acc + tv * wv

            acc = lax.fori_loop(0, A, step, jnp.zeros((16,), jnp.float32),
                                unroll=4)
            out_v[pl.ds(ch * CHUNK_ROWS + g * 16, 16)] = acc
            return 0

        lax.fori_loop(0, CHUNK_ROWS // 16, group, 0)

    pltpu.sync_copy(out_v, out_hbm.at[pl.ds(row0, ROWS_PER_TILE)])


OUT_PER_TILE = NC // NW
GATHER_SEG = 128


@functools.partial(
    pl.kernel,
    out_type=jax.ShapeDtypeStruct((B * C,), jnp.float32),
    mesh=_mesh,
    scratch_types=[
        pltpu.VMEM((OUT_PER_TILE,), jnp.int32),
        pltpu.VMEM((OUT_PER_TILE,), jnp.float32),
        pltpu.VMEM((OUT_PER_TILE,), jnp.float32),
        pltpu.VMEM((OUT_PER_TILE,), jnp.float32),
        pltpu.SemaphoreType.DMA,
    ],
    compiler_params=_params,
)
def _unflatten(scores_hbm, u_hbm, m_hbm, out_hbm,
               idx_v, g_v, m_v, o_v, sem):
    wid = _wid()
    p0 = wid * OUT_PER_TILE
    pltpu.sync_copy(u_hbm.at[pl.ds(p0, OUT_PER_TILE)], idx_v)
    pltpu.sync_copy(m_hbm.at[pl.ds(p0, OUT_PER_TILE)], m_v)

    nseg = OUT_PER_TILE // GATHER_SEG
    copies = [
        pltpu.make_async_copy(
            scores_hbm.at[idx_v.at[pl.ds(j * GATHER_SEG, GATHER_SEG)]],
            g_v.at[pl.ds(j * GATHER_SEG, GATHER_SEG)],
            sem)
        for j in range(nseg)
    ]
    for cp in copies:
        cp.start()
    for cp in copies:
        cp.wait()

    def step(k, _):
        s = pl.ds(k * 16, 16)
        o_v[s] = g_v[s] * m_v[s]
        return 0

    lax.fori_loop(0, OUT_PER_TILE // 16, step, 0)
    pltpu.sync_copy(o_v, out_hbm.at[pl.ds(p0, OUT_PER_TILE)])


def kernel(input_scores, alignments, alignment_weights, unflatten,
           unflatten_mask):
    tbl = jnp.reshape(input_scores, (-1,))
    ali = jnp.reshape(alignments.astype(jnp.int32), (-1,))
    w = jnp.reshape(alignment_weights, (-1,))
    scores_flat = _score_rows(tbl, ali, w)
    u = jnp.reshape(unflatten.astype(jnp.int32), (-1,))
    m = jnp.reshape(unflatten_mask, (-1,))
    out = _unflatten(scores_flat, u, m)
    return jnp.reshape(out, (B, C))


# speedup vs baseline: 228.6571x; 1.5385x over previous
import functools

import jax
import jax.numpy as jnp
from jax import lax
from jax.experimental import pallas as pl
from jax.experimental.pallas import tpu as pltpu
from jax.experimental.pallas import tpu_sc as plsc

B = 16
L = 4096
NC = 32768
A = 64
C = 2048

NUM_CORES = 2
NUM_SUBCORES = 16
NW = NUM_CORES * NUM_SUBCORES
ROWS_PER_TILE = NC // NW
CHUNK_ROWS = 128
CHUNK_ELEMS = CHUNK_ROWS * A
NUM_CHUNKS = ROWS_PER_TILE // CHUNK_ROWS
TABLE = B * L

_mesh = plsc.VectorSubcoreMesh(
    core_axis_name="c", subcore_axis_name="s",
    num_cores=NUM_CORES, num_subcores=NUM_SUBCORES)
_params = pltpu.CompilerParams(needs_layout_passes=False)


def _wid():
    return lax.axis_index("s") * NUM_CORES + lax.axis_index("c")


@functools.partial(
    pl.kernel,
    out_type=jax.ShapeDtypeStruct((NC,), jnp.float32),
    mesh=_mesh,
    scratch_types=[
        pltpu.VMEM((TABLE,), jnp.float32),
        pltpu.VMEM((CHUNK_ELEMS,), jnp.int32),
        pltpu.VMEM((CHUNK_ELEMS,), jnp.int32),
        pltpu.VMEM((CHUNK_ELEMS,), jnp.float32),
        pltpu.VMEM((CHUNK_ELEMS,), jnp.float32),
        pltpu.VMEM((ROWS_PER_TILE,), jnp.float32),
        pltpu.SemaphoreType.DMA,
        pltpu.SemaphoreType.DMA,
        pltpu.SemaphoreType.DMA,
    ],
    compiler_params=_params,
)
def _score_rows(tbl_hbm, ali_hbm, w_hbm, out_hbm,
                table_v, ib0, ib1, wb0, wb1, out_v, sem_t, sem0, sem1):
    wid = _wid()
    row0 = wid * ROWS_PER_TILE
    e0 = row0 * A

    tcp = pltpu.make_async_copy(tbl_hbm, table_v, sem_t)
    tcp.start()

    ibufs = (ib0, ib1)
    wbufs = (wb0, wb1)
    sems = (sem0, sem1)

    def chunk_copies(ch):
        b = ch % 2
        src = pl.ds(e0 + ch * CHUNK_ELEMS, CHUNK_ELEMS)
        return (pltpu.make_async_copy(ali_hbm.at[src], ibufs[b], sems[b]),
                pltpu.make_async_copy(w_hbm.at[src], wbufs[b], sems[b]))

    for cp in chunk_copies(0):
        cp.start()
    tcp.wait()

    lane = lax.iota(jnp.int32, 16)

    for ch in range(NUM_CHUNKS):
        for cp in chunk_copies(ch):
            cp.wait()
        if ch + 1 < NUM_CHUNKS:
            for cp in chunk_copies(ch + 1):
                cp.start()
        ib = ibufs[ch % 2]
        wb = wbufs[ch % 2]

        def group(g, _, ib=ib, wb=wb, ch=ch):
            gbase = g * (16 * A)

            def row(j, outacc):
                base = gbase + j * A
                acc = jnp.zeros((16,), jnp.float32)
                for q in range(A // 16):
                    s = pl.ds(base + q * 16, 16)
                    iv = ib[s]
                    wv = wb[s]
                    tv = plsc.load_gather(table_v, [iv])
                    acc = acc + tv * wv
                return jnp.where(lane == j, jnp.sum(acc), outacc)

            outacc = lax.fori_loop(0, 16, row, jnp.zeros((16,), jnp.float32),
                                   unroll=4)
            out_v[pl.ds(ch * CHUNK_ROWS + g * 16, 16)] = outacc
            return 0

        lax.fori_loop(0, CHUNK_ROWS // 16, group, 0)

    pltpu.sync_copy(out_v, out_hbm.at[pl.ds(row0, ROWS_PER_TILE)])


OUT_PER_TILE = NC // NW
GATHER_SEG = 128


@functools.partial(
    pl.kernel,
    out_type=jax.ShapeDtypeStruct((B * C,), jnp.float32),
    mesh=_mesh,
    scratch_types=[
        pltpu.VMEM((OUT_PER_TILE,), jnp.int32),
        pltpu.VMEM((OUT_PER_TILE,), jnp.float32),
        pltpu.VMEM((OUT_PER_TILE,), jnp.float32),
        pltpu.VMEM((OUT_PER_TILE,), jnp.float32),
        pltpu.SemaphoreType.DMA,
    ],
    compiler_params=_params,
)
def _unflatten(scores_hbm, u_hbm, m_hbm, out_hbm,
               idx_v, g_v, m_v, o_v, sem):
    wid = _wid()
    p0 = wid * OUT_PER_TILE
    pltpu.sync_copy(u_hbm.at[pl.ds(p0, OUT_PER_TILE)], idx_v)
    pltpu.sync_copy(m_hbm.at[pl.ds(p0, OUT_PER_TILE)], m_v)

    nseg = OUT_PER_TILE // GATHER_SEG
    copies = [
        pltpu.make_async_copy(
            scores_hbm.at[idx_v.at[pl.ds(j * GATHER_SEG, GATHER_SEG)]],
            g_v.at[pl.ds(j * GATHER_SEG, GATHER_SEG)],
            sem)
        for j in range(nseg)
    ]
    for cp in copies:
        cp.start()
    for cp in copies:
        cp.wait()

    def step(k, _):
        s = pl.ds(k * 16, 16)
        o_v[s] = g_v[s] * m_v[s]
        return 0

    lax.fori_loop(0, OUT_PER_TILE // 16, step, 0)
    pltpu.sync_copy(o_v, out_hbm.at[pl.ds(p0, OUT_PER_TILE)])


def kernel(input_scores, alignments, alignment_weights, unflatten,
           unflatten_mask):
    tbl = jnp.reshape(input_scores, (-1,))
    ali = jnp.reshape(alignments.astype(jnp.int32), (-1,))
    w = jnp.reshape(alignment_weights, (-1,))
    scores_flat = _score_rows(tbl, ali, w)
    u = jnp.reshape(unflatten.astype(jnp.int32), (-1,))
    m = jnp.reshape(unflatten_mask, (-1,))
    out = _unflatten(scores_flat, u, m)
    return jnp.reshape(out, (B, C))
